# baseline (speedup 1.0000x reference)
import functools

import jax
import jax.numpy as jnp
from jax import lax
from jax.experimental import pallas as pl
from jax.experimental.pallas import tpu as pltpu
from jax.experimental.pallas import tpu_sc as plsc

B, N, C, D, K = 8, 4096, 64, 128, 32
M = N // 4
QT = 256
NW = 32
QW = (B * M) // NW
CQ = 4
NCH = QW // CQ
IPW = QW * K // 128


def _fps_body(pos_ref, out_ref, dist_ref):
    x = pos_ref[0]
    y = pos_ref[1]
    z = pos_ref[2]
    lane = jax.lax.broadcasted_iota(jnp.int32, (B, N), 1)
    lane_m = jax.lax.broadcasted_iota(jnp.int32, (B, M), 1)
    dist_ref[...] = jnp.full((B, N), 1e10, jnp.float32)

    def body(i, first):
        oh = lane == first
        cx = jnp.sum(jnp.where(oh, x, 0.0), axis=1, keepdims=True)
        cy = jnp.sum(jnp.where(oh, y, 0.0), axis=1, keepdims=True)
        cz = jnp.sum(jnp.where(oh, z, 0.0), axis=1, keepdims=True)
        sel = lane_m == i
        out_ref[0] = jnp.where(sel, cx, out_ref[0])
        out_ref[1] = jnp.where(sel, cy, out_ref[1])
        out_ref[2] = jnp.where(sel, cz, out_ref[2])
        dx = x - cx
        dy = y - cy
        dz = z - cz
        d = (dx * dx + dy * dy) + dz * dz
        dist = jnp.minimum(dist_ref[...], d)
        dist_ref[...] = dist
        mx = jnp.max(dist, axis=1, keepdims=True)
        return jnp.min(jnp.where(dist == mx, lane, N), axis=1, keepdims=True)

    jax.lax.fori_loop(0, M, body, jnp.zeros((B, 1), jnp.int32))


def _h_body(f_ref, w_ref, b_ref, h_ref):
    h = jnp.dot(f_ref[0], w_ref[...], preferred_element_type=jnp.float32)
    h_ref[0] = jnp.maximum(h + b_ref[...], 0.0)


def _knn_body(pos_ref, q_ref, ind_ref):
    bi = pl.program_id(0)
    x = pos_ref[0, 0:1]
    y = pos_ref[0, 1:2]
    z = pos_ref[0, 2:3]
    lane8 = jax.lax.broadcasted_iota(jnp.int32, (QT, B), 1)
    bsel = lane8 == bi
    qx = jnp.sum(jnp.where(bsel, q_ref[0], 0.0), axis=1, keepdims=True)
    qy = jnp.sum(jnp.where(bsel, q_ref[1], 0.0), axis=1, keepdims=True)
    qz = jnp.sum(jnp.where(bsel, q_ref[2], 0.0), axis=1, keepdims=True)
    dx = qx - x
    dy = qy - y
    dz = qz - z
    lane = jax.lax.broadcasted_iota(jnp.int32, (QT, N), 1)
    lane_k = jax.lax.broadcasted_iota(jnp.int32, (QT, K), 1)

    def step(j, carry):
        d, ind = carry
        m = jnp.min(d, axis=1, keepdims=True)
        first = jnp.min(jnp.where(d == m, lane, N), axis=1, keepdims=True)
        ind = jnp.where(lane_k == j, first + bi * N, ind)
        d = jnp.where(lane == first, jnp.inf, d)
        return d, ind

    d0 = (dx * dx + dy * dy) + dz * dz
    _, ind = jax.lax.fori_loop(
        0, K, step, (d0, jnp.zeros((QT, K), jnp.int32)))
    ind_ref[0] = ind


def _sc_gather_body(h_hbm, ind_hbm, out_hbm, idx_v, rows0, rows1, outb,
                    sem0, sem1, osem):
    wid = lax.axis_index("s") * 2 + lax.axis_index("c")
    pltpu.sync_copy(ind_hbm.at[pl.ds(wid * IPW, IPW)], idx_v)
    pltpu.async_copy(h_hbm.at[idx_v.at[0]], rows0, sem0)

    def compute(rows_b, c):
        for q in range(CQ):
            base = q * K
            vecs = tuple(rows_b[base, pl.ds(dv * 16, 16)] for dv in range(8))

            def kstep(k, vs):
                return tuple(
                    jnp.maximum(vs[dv], rows_b[base + k, pl.ds(dv * 16, 16)])
                    for dv in range(8))

            vecs = lax.fori_loop(1, K, kstep, vecs)
            for dv in range(8):
                outb[q, pl.ds(dv * 16, 16)] = vecs[dv]
        pltpu.async_copy(
            outb, out_hbm.at[pl.ds(wid * QW + c * CQ, CQ)], osem).wait()

    def pair(p, _):
        c0 = 2 * p
        c1 = 2 * p + 1
        pltpu.async_copy(h_hbm.at[idx_v.at[c1]], rows1, sem1)
        pltpu.make_async_copy(h_hbm.at[idx_v.at[c0]], rows0, sem0).wait()
        compute(rows0, c0)

        @pl.when(c1 + 1 < NCH)
        def _():
            pltpu.async_copy(h_hbm.at[idx_v.at[c1 + 1]], rows0, sem0)

        pltpu.make_async_copy(h_hbm.at[idx_v.at[c1]], rows1, sem1).wait()
        compute(rows1, c1)
        return 0

    lax.fori_loop(0, NCH // 2, pair, 0)


@functools.cache
def _sc_gather():
    return pl.kernel(
        _sc_gather_body,
        mesh=plsc.VectorSubcoreMesh(core_axis_name="c", subcore_axis_name="s"),
        out_type=jax.ShapeDtypeStruct((B * M, D), jnp.float32),
        scratch_types=[
            pltpu.VMEM((IPW, 128), jnp.int32),
            pltpu.VMEM((CQ * K, D), jnp.float32),
            pltpu.VMEM((CQ * K, D), jnp.float32),
            pltpu.VMEM((CQ, D), jnp.float32),
            pltpu.SemaphoreType.DMA,
            pltpu.SemaphoreType.DMA,
            pltpu.SemaphoreType.DMA,
        ],
    )


def kernel(feature, pos, W, b):
    pos_t = jnp.transpose(pos, (2, 0, 1))

    sampled_c = pl.pallas_call(
        _fps_body,
        out_shape=jax.ShapeDtypeStruct((3, B, M), jnp.float32),
        scratch_shapes=[pltpu.VMEM((B, N), jnp.float32)],
    )(pos_t)

    sampled_pos = jnp.transpose(sampled_c, (1, 2, 0))
    q_cols = jnp.transpose(sampled_c, (0, 2, 1))

    h = pl.pallas_call(
        _h_body,
        grid=(B,),
        in_specs=[
            pl.BlockSpec((1, N, C), lambda bi: (bi, 0, 0)),
            pl.BlockSpec((C, D), lambda bi: (0, 0)),
            pl.BlockSpec((1, D), lambda bi: (0, 0)),
        ],
        out_specs=pl.BlockSpec((1, N, D), lambda bi: (bi, 0, 0)),
        out_shape=jax.ShapeDtypeStruct((B, N, D), jnp.float32),
    )(feature, W, b.reshape(1, D))

    ind = pl.pallas_call(
        _knn_body,
        grid=(B, M // QT),
        in_specs=[
            pl.BlockSpec((1, 3, N), lambda bi, qi: (bi, 0, 0)),
            pl.BlockSpec((3, QT, B), lambda bi, qi: (0, qi, 0)),
        ],
        out_specs=pl.BlockSpec((1, QT, K), lambda bi, qi: (bi, qi, 0)),
        out_shape=jax.ShapeDtypeStruct((B, M, K), jnp.int32),
    )(jnp.transpose(pos, (0, 2, 1)), q_cols)

    out = _sc_gather()(h.reshape(B * N, D),
                       ind.reshape((B * M * K) // 128, 128))
    return (out.reshape(B, M, D), sampled_pos)


# speedup vs baseline: 1.8551x; 1.8551x over previous
import functools

import jax
import jax.numpy as jnp
from jax import lax
from jax.experimental import pallas as pl
from jax.experimental.pallas import tpu as pltpu
from jax.experimental.pallas import tpu_sc as plsc

B, N, C, D, K = 8, 4096, 64, 128, 32
M = N // 4
QT = 256
NW = 32
QW = (B * M) // NW
CQ = 4
NCH = QW // CQ
IPW = QW * K // 128


def _fps_body(pos_ref, out_ref, dist_ref):
    x = pos_ref[0]
    y = pos_ref[1]
    z = pos_ref[2]
    lane = jax.lax.broadcasted_iota(jnp.int32, (B, N), 1)
    lane_m = jax.lax.broadcasted_iota(jnp.int32, (B, M), 1)
    dist_ref[...] = jnp.full((B, N), 1e10, jnp.float32)

    def body(i, first):
        oh = lane == first
        cx = jnp.sum(jnp.where(oh, x, 0.0), axis=1, keepdims=True)
        cy = jnp.sum(jnp.where(oh, y, 0.0), axis=1, keepdims=True)
        cz = jnp.sum(jnp.where(oh, z, 0.0), axis=1, keepdims=True)
        sel = lane_m == i
        out_ref[0] = jnp.where(sel, cx, out_ref[0])
        out_ref[1] = jnp.where(sel, cy, out_ref[1])
        out_ref[2] = jnp.where(sel, cz, out_ref[2])
        dx = x - cx
        dy = y - cy
        dz = z - cz
        d = (dx * dx + dy * dy) + dz * dz
        dist = jnp.minimum(dist_ref[...], d)
        dist_ref[...] = dist
        mx = jnp.max(dist, axis=1, keepdims=True)
        return jnp.min(jnp.where(dist == mx, lane, N), axis=1, keepdims=True)

    jax.lax.fori_loop(0, M, body, jnp.zeros((B, 1), jnp.int32))


def _h_body(f_ref, w_ref, b_ref, h_ref):
    h = jnp.dot(f_ref[0], w_ref[...], preferred_element_type=jnp.float32)
    h_ref[0] = jnp.maximum(h + b_ref[...], 0.0)


def _knn_body(pos_ref, q_ref, ind_ref, key_ref):
    bi = pl.program_id(0)
    x = pos_ref[0, 0:1]
    y = pos_ref[0, 1:2]
    z = pos_ref[0, 2:3]
    lane8 = jax.lax.broadcasted_iota(jnp.int32, (QT, B), 1)
    bsel = lane8 == bi
    qx = jnp.sum(jnp.where(bsel, q_ref[0], 0.0), axis=1, keepdims=True)
    qy = jnp.sum(jnp.where(bsel, q_ref[1], 0.0), axis=1, keepdims=True)
    qz = jnp.sum(jnp.where(bsel, q_ref[2], 0.0), axis=1, keepdims=True)
    dx = qx - x
    dy = qy - y
    dz = qz - z
    lane = jax.lax.broadcasted_iota(jnp.int32, (QT, N), 1)
    lane_k = jax.lax.broadcasted_iota(jnp.int32, (QT, K), 1)

    d0 = (dx * dx + dy * dy) + dz * dz
    key_ref[...] = (jax.lax.bitcast_convert_type(d0, jnp.int32)
                    & jnp.int32(-4096)) | lane

    def step(j, ind):
        key = key_ref[...]
        m = jnp.min(key, axis=1, keepdims=True)
        ind = jnp.where(lane_k == j, (m & 4095) + bi * N, ind)
        key_ref[...] = jnp.where(key == m, jnp.int32(0x7FFFFFFF), key)
        return ind

    ind_ref[0] = jax.lax.fori_loop(
        0, K, step, jnp.zeros((QT, K), jnp.int32))


def _sc_gather_body(h_hbm, ind_hbm, out_hbm, idx_v, rows0, rows1, outb,
                    sem0, sem1, osem):
    wid = lax.axis_index("s") * 2 + lax.axis_index("c")
    pltpu.sync_copy(ind_hbm.at[pl.ds(wid * IPW, IPW)], idx_v)
    pltpu.async_copy(h_hbm.at[idx_v.at[0]], rows0, sem0)

    def compute(rows_b, c):
        for q in range(CQ):
            base = q * K
            vecs = tuple(rows_b[base, pl.ds(dv * 16, 16)] for dv in range(8))

            def kstep(k, vs):
                return tuple(
                    jnp.maximum(vs[dv], rows_b[base + k, pl.ds(dv * 16, 16)])
                    for dv in range(8))

            vecs = lax.fori_loop(1, K, kstep, vecs)
            for dv in range(8):
                outb[q, pl.ds(dv * 16, 16)] = vecs[dv]
        pltpu.async_copy(
            outb, out_hbm.at[pl.ds(wid * QW + c * CQ, CQ)], osem).wait()

    def pair(p, _):
        c0 = 2 * p
        c1 = 2 * p + 1
        pltpu.async_copy(h_hbm.at[idx_v.at[c1]], rows1, sem1)
        pltpu.make_async_copy(h_hbm.at[idx_v.at[c0]], rows0, sem0).wait()
        compute(rows0, c0)

        @pl.when(c1 + 1 < NCH)
        def _():
            pltpu.async_copy(h_hbm.at[idx_v.at[c1 + 1]], rows0, sem0)

        pltpu.make_async_copy(h_hbm.at[idx_v.at[c1]], rows1, sem1).wait()
        compute(rows1, c1)
        return 0

    lax.fori_loop(0, NCH // 2, pair, 0)


@functools.cache
def _sc_gather():
    return pl.kernel(
        _sc_gather_body,
        mesh=plsc.VectorSubcoreMesh(core_axis_name="c", subcore_axis_name="s"),
        out_type=jax.ShapeDtypeStruct((B * M, D), jnp.float32),
        scratch_types=[
            pltpu.VMEM((IPW, 128), jnp.int32),
            pltpu.VMEM((CQ * K, D), jnp.float32),
            pltpu.VMEM((CQ * K, D), jnp.float32),
            pltpu.VMEM((CQ, D), jnp.float32),
            pltpu.SemaphoreType.DMA,
            pltpu.SemaphoreType.DMA,
            pltpu.SemaphoreType.DMA,
        ],
    )


def kernel(feature, pos, W, b):
    pos_t = jnp.transpose(pos, (2, 0, 1))

    sampled_c = pl.pallas_call(
        _fps_body,
        out_shape=jax.ShapeDtypeStruct((3, B, M), jnp.float32),
        scratch_shapes=[pltpu.VMEM((B, N), jnp.float32)],
    )(pos_t)

    sampled_pos = jnp.transpose(sampled_c, (1, 2, 0))
    q_cols = jnp.transpose(sampled_c, (0, 2, 1))

    h = pl.pallas_call(
        _h_body,
        grid=(B,),
        in_specs=[
            pl.BlockSpec((1, N, C), lambda bi: (bi, 0, 0)),
            pl.BlockSpec((C, D), lambda bi: (0, 0)),
            pl.BlockSpec((1, D), lambda bi: (0, 0)),
        ],
        out_specs=pl.BlockSpec((1, N, D), lambda bi: (bi, 0, 0)),
        out_shape=jax.ShapeDtypeStruct((B, N, D), jnp.float32),
    )(feature, W, b.reshape(1, D))

    ind = pl.pallas_call(
        _knn_body,
        grid=(B, M // QT),
        in_specs=[
            pl.BlockSpec((1, 3, N), lambda bi, qi: (bi, 0, 0)),
            pl.BlockSpec((3, QT, B), lambda bi, qi: (0, qi, 0)),
        ],
        out_specs=pl.BlockSpec((1, QT, K), lambda bi, qi: (bi, qi, 0)),
        out_shape=jax.ShapeDtypeStruct((B, M, K), jnp.int32),
        scratch_shapes=[pltpu.VMEM((QT, N), jnp.int32)],
    )(jnp.transpose(pos, (0, 2, 1)), q_cols)

    out = _sc_gather()(h.reshape(B * N, D),
                       ind.reshape((B * M * K) // 128, 128))
    return (out.reshape(B, M, D), sampled_pos)


# speedup vs baseline: 1.8692x; 1.0076x over previous
import functools

import jax
import jax.numpy as jnp
from jax import lax
from jax.experimental import pallas as pl
from jax.experimental.pallas import tpu as pltpu
from jax.experimental.pallas import tpu_sc as plsc

B, N, C, D, K = 8, 4096, 64, 128, 32
M = N // 4
QT = 256
NW = 32
QW = (B * M) // NW
CQ = 4
NCH = QW // CQ
IPW = QW * K // 128


def _fps_body(pos_ref, out_ref, dist_ref):
    x = pos_ref[0]
    y = pos_ref[1]
    z = pos_ref[2]
    lane = jax.lax.broadcasted_iota(jnp.int32, (B, N), 1)
    lane_m = jax.lax.broadcasted_iota(jnp.int32, (B, M), 1)
    dist_ref[...] = jnp.full((B, N), 1e10, jnp.float32)

    def body(i, first):
        oh = lane == first
        cx = jnp.sum(jnp.where(oh, x, 0.0), axis=1, keepdims=True)
        cy = jnp.sum(jnp.where(oh, y, 0.0), axis=1, keepdims=True)
        cz = jnp.sum(jnp.where(oh, z, 0.0), axis=1, keepdims=True)
        sel = lane_m == i
        out_ref[0] = jnp.where(sel, cx, out_ref[0])
        out_ref[1] = jnp.where(sel, cy, out_ref[1])
        out_ref[2] = jnp.where(sel, cz, out_ref[2])
        dx = x - cx
        dy = y - cy
        dz = z - cz
        d = (dx * dx + dy * dy) + dz * dz
        dist = jnp.minimum(dist_ref[...], d)
        dist_ref[...] = dist
        mx = jnp.max(dist, axis=1, keepdims=True)
        return jnp.min(jnp.where(dist == mx, lane, N), axis=1, keepdims=True)

    jax.lax.fori_loop(0, M, body, jnp.zeros((B, 1), jnp.int32))


def _h_body(f_ref, w_ref, b_ref, h_ref):
    h = jnp.dot(f_ref[0], w_ref[...], preferred_element_type=jnp.float32)
    h_ref[0] = jnp.maximum(h + b_ref[...], 0.0)


def _knn_body(pos_ref, q_ref, ind_ref, key_ref):
    bi = pl.program_id(0)
    x = pos_ref[0, 0:1]
    y = pos_ref[0, 1:2]
    z = pos_ref[0, 2:3]
    lane8 = jax.lax.broadcasted_iota(jnp.int32, (QT, B), 1)
    bsel = lane8 == bi
    qx = jnp.sum(jnp.where(bsel, q_ref[0], 0.0), axis=1, keepdims=True)
    qy = jnp.sum(jnp.where(bsel, q_ref[1], 0.0), axis=1, keepdims=True)
    qz = jnp.sum(jnp.where(bsel, q_ref[2], 0.0), axis=1, keepdims=True)
    dx = qx - x
    dy = qy - y
    dz = qz - z
    lane = jax.lax.broadcasted_iota(jnp.int32, (QT, N), 1)
    lane_k = jax.lax.broadcasted_iota(jnp.int32, (QT, K), 1)

    d0 = (dx * dx + dy * dy) + dz * dz
    key_ref[...] = (jax.lax.bitcast_convert_type(d0, jnp.int32)
                    & jnp.int32(-4096)) | lane

    def step(j, carry):
        m_prev, ind = carry
        key = key_ref[...]
        m = jnp.min(jnp.where(key > m_prev, key, jnp.int32(0x7FFFFFFF)),
                    axis=1, keepdims=True)
        ind = jnp.where(lane_k == j, (m & 4095) + bi * N, ind)
        return m, ind

    _, ind = jax.lax.fori_loop(
        0, K, step,
        (jnp.full((QT, 1), -1, jnp.int32), jnp.zeros((QT, K), jnp.int32)))
    ind_ref[0] = ind


def _sc_gather_body(h_hbm, ind_hbm, out_hbm, idx_v, rows0, rows1, outb,
                    sem0, sem1, osem):
    wid = lax.axis_index("s") * 2 + lax.axis_index("c")
    pltpu.sync_copy(ind_hbm.at[pl.ds(wid * IPW, IPW)], idx_v)
    pltpu.async_copy(h_hbm.at[idx_v.at[0]], rows0, sem0)

    def compute(rows_b, c):
        for q in range(CQ):
            base = q * K
            vecs = tuple(rows_b[base, pl.ds(dv * 16, 16)] for dv in range(8))

            def kstep(k, vs):
                return tuple(
                    jnp.maximum(vs[dv], rows_b[base + k, pl.ds(dv * 16, 16)])
                    for dv in range(8))

            vecs = lax.fori_loop(1, K, kstep, vecs)
            for dv in range(8):
                outb[q, pl.ds(dv * 16, 16)] = vecs[dv]
        pltpu.async_copy(
            outb, out_hbm.at[pl.ds(wid * QW + c * CQ, CQ)], osem).wait()

    def pair(p, _):
        c0 = 2 * p
        c1 = 2 * p + 1
        pltpu.async_copy(h_hbm.at[idx_v.at[c1]], rows1, sem1)
        pltpu.make_async_copy(h_hbm.at[idx_v.at[c0]], rows0, sem0).wait()
        compute(rows0, c0)

        @pl.when(c1 + 1 < NCH)
        def _():
            pltpu.async_copy(h_hbm.at[idx_v.at[c1 + 1]], rows0, sem0)

        pltpu.make_async_copy(h_hbm.at[idx_v.at[c1]], rows1, sem1).wait()
        compute(rows1, c1)
        return 0

    lax.fori_loop(0, NCH // 2, pair, 0)


@functools.cache
def _sc_gather():
    return pl.kernel(
        _sc_gather_body,
        mesh=plsc.VectorSubcoreMesh(core_axis_name="c", subcore_axis_name="s"),
        out_type=jax.ShapeDtypeStruct((B * M, D), jnp.float32),
        scratch_types=[
            pltpu.VMEM((IPW, 128), jnp.int32),
            pltpu.VMEM((CQ * K, D), jnp.float32),
            pltpu.VMEM((CQ * K, D), jnp.float32),
            pltpu.VMEM((CQ, D), jnp.float32),
            pltpu.SemaphoreType.DMA,
            pltpu.SemaphoreType.DMA,
            pltpu.SemaphoreType.DMA,
        ],
    )


def kernel(feature, pos, W, b):
    pos_t = jnp.transpose(pos, (2, 0, 1))

    sampled_c = pl.pallas_call(
        _fps_body,
        out_shape=jax.ShapeDtypeStruct((3, B, M), jnp.float32),
        scratch_shapes=[pltpu.VMEM((B, N), jnp.float32)],
    )(pos_t)

    sampled_pos = jnp.transpose(sampled_c, (1, 2, 0))
    q_cols = jnp.transpose(sampled_c, (0, 2, 1))

    h = pl.pallas_call(
        _h_body,
        grid=(B,),
        in_specs=[
            pl.BlockSpec((1, N, C), lambda bi: (bi, 0, 0)),
            pl.BlockSpec((C, D), lambda bi: (0, 0)),
            pl.BlockSpec((1, D), lambda bi: (0, 0)),
        ],
        out_specs=pl.BlockSpec((1, N, D), lambda bi: (bi, 0, 0)),
        out_shape=jax.ShapeDtypeStruct((B, N, D), jnp.float32),
    )(feature, W, b.reshape(1, D))

    ind = pl.pallas_call(
        _knn_body,
        grid=(B, M // QT),
        in_specs=[
            pl.BlockSpec((1, 3, N), lambda bi, qi: (bi, 0, 0)),
            pl.BlockSpec((3, QT, B), lambda bi, qi: (0, qi, 0)),
        ],
        out_specs=pl.BlockSpec((1, QT, K), lambda bi, qi: (bi, qi, 0)),
        out_shape=jax.ShapeDtypeStruct((B, M, K), jnp.int32),
        scratch_shapes=[pltpu.VMEM((QT, N), jnp.int32)],
    )(jnp.transpose(pos, (0, 2, 1)), q_cols)

    out = _sc_gather()(h.reshape(B * N, D),
                       ind.reshape((B * M * K) // 128, 128))
    return (out.reshape(B, M, D), sampled_pos)


# speedup vs baseline: 1.9754x; 1.0569x over previous
import functools

import jax
import jax.numpy as jnp
from jax import lax
from jax.experimental import pallas as pl
from jax.experimental.pallas import tpu as pltpu
from jax.experimental.pallas import tpu_sc as plsc

B, N, C, D, K = 8, 4096, 64, 128, 32
M = N // 4
QT = 512
NW = 32
QW = (B * M) // NW
CQ = 4
NCH = QW // CQ
IPW = QW * K // 128


def _fps_body(pos_ref, out_ref, dist_ref):
    x = pos_ref[0]
    y = pos_ref[1]
    z = pos_ref[2]
    lane = jax.lax.broadcasted_iota(jnp.int32, (B, N), 1)
    lane_m = jax.lax.broadcasted_iota(jnp.int32, (B, M), 1)
    dist_ref[...] = jnp.full((B, N), 1e10, jnp.float32)

    def body(i, first):
        oh = lane == first
        cx = jnp.sum(jnp.where(oh, x, 0.0), axis=1, keepdims=True)
        cy = jnp.sum(jnp.where(oh, y, 0.0), axis=1, keepdims=True)
        cz = jnp.sum(jnp.where(oh, z, 0.0), axis=1, keepdims=True)
        sel = lane_m == i
        out_ref[0] = jnp.where(sel, cx, out_ref[0])
        out_ref[1] = jnp.where(sel, cy, out_ref[1])
        out_ref[2] = jnp.where(sel, cz, out_ref[2])
        dx = x - cx
        dy = y - cy
        dz = z - cz
        d = (dx * dx + dy * dy) + dz * dz
        dist = jnp.minimum(dist_ref[...], d)
        dist_ref[...] = dist
        mx = jnp.max(dist, axis=1, keepdims=True)
        return jnp.min(jnp.where(dist == mx, lane, N), axis=1, keepdims=True)

    jax.lax.fori_loop(0, M, body, jnp.zeros((B, 1), jnp.int32))


def _knn_body(pos_ref, q_ref, f_ref, w_ref, b_ref, ind_ref, h_ref, key_ref):
    bi = pl.program_id(0)
    qi = pl.program_id(1)

    @pl.when(qi == 0)
    def _():
        h = jnp.dot(f_ref[0], w_ref[...], preferred_element_type=jnp.float32)
        h_ref[0] = jnp.maximum(h + b_ref[...], 0.0)

    x = pos_ref[0, 0:1]
    y = pos_ref[0, 1:2]
    z = pos_ref[0, 2:3]
    lane8 = jax.lax.broadcasted_iota(jnp.int32, (QT, B), 1)
    bsel = lane8 == bi
    qx = jnp.sum(jnp.where(bsel, q_ref[0], 0.0), axis=1, keepdims=True)
    qy = jnp.sum(jnp.where(bsel, q_ref[1], 0.0), axis=1, keepdims=True)
    qz = jnp.sum(jnp.where(bsel, q_ref[2], 0.0), axis=1, keepdims=True)
    dx = qx - x
    dy = qy - y
    dz = qz - z
    lane = jax.lax.broadcasted_iota(jnp.int32, (QT, N), 1)
    lane_k = jax.lax.broadcasted_iota(jnp.int32, (QT, K), 1)

    d0 = (dx * dx + dy * dy) + dz * dz
    key_ref[...] = (jax.lax.bitcast_convert_type(d0, jnp.int32)
                    & jnp.int32(-4096)) | lane

    def step(j, carry):
        m_prev, ind = carry
        key = key_ref[...]
        m = jnp.min(jnp.where(key > m_prev, key, jnp.int32(0x7FFFFFFF)),
                    axis=1, keepdims=True)
        ind = jnp.where(lane_k == j, (m & 4095) + bi * N, ind)
        return m, ind

    _, ind = jax.lax.fori_loop(
        0, K, step,
        (jnp.full((QT, 1), -1, jnp.int32), jnp.zeros((QT, K), jnp.int32)))
    ind_ref[0] = ind


def _sc_gather_body(h_hbm, ind_hbm, out_hbm, idx_v, rows0, rows1, outb,
                    sem0, sem1, osem):
    wid = lax.axis_index("s") * 2 + lax.axis_index("c")
    pltpu.sync_copy(ind_hbm.at[pl.ds(wid * IPW, IPW)], idx_v)
    pltpu.async_copy(h_hbm.at[idx_v.at[0]], rows0, sem0)

    def compute(rows_b, c):
        for q in range(CQ):
            base = q * K
            vecs = tuple(rows_b[base, pl.ds(dv * 16, 16)] for dv in range(8))

            def kstep(k, vs):
                return tuple(
                    jnp.maximum(vs[dv], rows_b[base + k, pl.ds(dv * 16, 16)])
                    for dv in range(8))

            vecs = lax.fori_loop(1, K, kstep, vecs)
            for dv in range(8):
                outb[q, pl.ds(dv * 16, 16)] = vecs[dv]
        pltpu.async_copy(
            outb, out_hbm.at[pl.ds(wid * QW + c * CQ, CQ)], osem).wait()

    def pair(p, _):
        c0 = 2 * p
        c1 = 2 * p + 1
        pltpu.async_copy(h_hbm.at[idx_v.at[c1]], rows1, sem1)
        pltpu.make_async_copy(h_hbm.at[idx_v.at[c0]], rows0, sem0).wait()
        compute(rows0, c0)

        @pl.when(c1 + 1 < NCH)
        def _():
            pltpu.async_copy(h_hbm.at[idx_v.at[c1 + 1]], rows0, sem0)

        pltpu.make_async_copy(h_hbm.at[idx_v.at[c1]], rows1, sem1).wait()
        compute(rows1, c1)
        return 0

    lax.fori_loop(0, NCH // 2, pair, 0)


@functools.cache
def _sc_gather():
    return pl.kernel(
        _sc_gather_body,
        mesh=plsc.VectorSubcoreMesh(core_axis_name="c", subcore_axis_name="s"),
        out_type=jax.ShapeDtypeStruct((B * M, D), jnp.float32),
        scratch_types=[
            pltpu.VMEM((IPW, 128), jnp.int32),
            pltpu.VMEM((CQ * K, D), jnp.float32),
            pltpu.VMEM((CQ * K, D), jnp.float32),
            pltpu.VMEM((CQ, D), jnp.float32),
            pltpu.SemaphoreType.DMA,
            pltpu.SemaphoreType.DMA,
            pltpu.SemaphoreType.DMA,
        ],
    )


def kernel(feature, pos, W, b):
    pos_t = jnp.transpose(pos, (2, 0, 1))

    sampled_c = pl.pallas_call(
        _fps_body,
        out_shape=jax.ShapeDtypeStruct((3, B, M), jnp.float32),
        scratch_shapes=[pltpu.VMEM((B, N), jnp.float32)],
    )(pos_t)

    sampled_pos = jnp.transpose(sampled_c, (1, 2, 0))
    q_cols = jnp.transpose(sampled_c, (0, 2, 1))

    ind, h = pl.pallas_call(
        _knn_body,
        grid=(B, M // QT),
        in_specs=[
            pl.BlockSpec((1, 3, N), lambda bi, qi: (bi, 0, 0)),
            pl.BlockSpec((3, QT, B), lambda bi, qi: (0, qi, 0)),
            pl.BlockSpec((1, N, C), lambda bi, qi: (bi, 0, 0)),
            pl.BlockSpec((C, D), lambda bi, qi: (0, 0)),
            pl.BlockSpec((1, D), lambda bi, qi: (0, 0)),
        ],
        out_specs=[
            pl.BlockSpec((1, QT, K), lambda bi, qi: (bi, qi, 0)),
            pl.BlockSpec((1, N, D), lambda bi, qi: (bi, 0, 0)),
        ],
        out_shape=[
            jax.ShapeDtypeStruct((B, M, K), jnp.int32),
            jax.ShapeDtypeStruct((B, N, D), jnp.float32),
        ],
        scratch_shapes=[pltpu.VMEM((QT, N), jnp.int32)],
    )(jnp.transpose(pos, (0, 2, 1)), q_cols, feature, W, b.reshape(1, D))

    out = _sc_gather()(h.reshape(B * N, D),
                       ind.reshape((B * M * K) // 128, 128))
    return (out.reshape(B, M, D), sampled_pos)
